# Initial kernel scaffold; baseline (speedup 1.0000x reference)
import jax
import jax.numpy as jnp
from jax.experimental import pallas as pl


def kernel(x, memory):
    raise NotImplementedError("write your pallas kernel here")


# speedup vs baseline: 1.0033x; 1.0033x over previous
import functools

import jax
import jax.numpy as jnp
from jax import lax
from jax.experimental import pallas as pl

B = 4
NUM_FEAT = 768
D1 = 24
D2 = 24
NMEM = 1024
MLEN = 16
GROUPS = NUM_FEAT // MLEN
SPATIAL = D1 * D2
NBLK = B * GROUPS


def _vq_block(x_ref, mem_ref, out_ref):
    X = x_ref[0]
    mem = mem_ref[...]
    nrm = jnp.sqrt(jnp.sum(mem * mem, axis=1, keepdims=True))
    mnorm = mem / jnp.maximum(nrm, 1e-12)
    xnrm = jnp.sqrt(jnp.sum(X * X, axis=0, keepdims=True))
    Xn = X / jnp.maximum(xnrm, 1e-12)
    sim = lax.dot_general(
        Xn, mnorm, (((0,), (1,)), ((), ())),
        preferred_element_type=jnp.float32)
    idx = jnp.argmax(sim, axis=1).reshape(SPATIAL, 1)
    iota = lax.broadcasted_iota(jnp.int32, (SPATIAL, NMEM), 1)
    onehot = (iota == idx).astype(jnp.float32)
    out = lax.dot_general(
        mem, onehot, (((0,), (1,)), ((), ())),
        precision=lax.Precision.HIGHEST,
        preferred_element_type=jnp.float32)
    out_ref[0] = out


@functools.partial(jax.jit, static_argnames=())
def kernel(x, memory):
    xr = x.reshape(NBLK, MLEN, SPATIAL)
    out = pl.pallas_call(
        _vq_block,
        grid=(NBLK,),
        in_specs=[
            pl.BlockSpec((1, MLEN, SPATIAL), lambda i: (i, 0, 0)),
            pl.BlockSpec((NMEM, MLEN), lambda i: (0, 0)),
        ],
        out_specs=pl.BlockSpec((1, MLEN, SPATIAL), lambda i: (i, 0, 0)),
        out_shape=jax.ShapeDtypeStruct((NBLK, MLEN, SPATIAL), jnp.float32),
    )(xr, memory)
    return out.reshape(B, NUM_FEAT, D1, D2)


# speedup vs baseline: 1.9020x; 1.8958x over previous
import functools

import jax
import jax.numpy as jnp
from jax import lax
from jax.experimental import pallas as pl
from jax.experimental.pallas import tpu as pltpu
from jax.experimental.pallas import tpu_sc as plsc

B = 4
NUM_FEAT = 768
D1 = 24
D2 = 24
NMEM = 1024
MLEN = 16
GROUPS = NUM_FEAT // MLEN
SPATIAL = D1 * D2
NBLK = B * GROUPS

NCORES = 2
NSUB = 16
NW = NCORES * NSUB
BLK_PER_W = NBLK // NW
LANES = 16
JSTEPS = SPATIAL // LANES


def _argmax_block(x_ref, mem_ref, idx_ref, mnorm_ref):
    @pl.when(pl.program_id(0) == 0)
    def _init():
        mem = mem_ref[...]
        nrm = jnp.sqrt(jnp.sum(mem * mem, axis=1, keepdims=True))
        mnorm_ref[...] = mem / jnp.maximum(nrm, 1e-12)

    X = x_ref[0]
    xnrm = jnp.sqrt(jnp.sum(X * X, axis=0, keepdims=True))
    Xn = X / jnp.maximum(xnrm, 1e-12)
    sim = lax.dot_general(
        Xn, mnorm_ref[...], (((0,), (1,)), ((), ())),
        preferred_element_type=jnp.float32)
    idx_ref[0] = jnp.argmax(sim, axis=1).reshape(1, SPATIAL)


def _tc_indices(xr, memory):
    return pl.pallas_call(
        _argmax_block,
        grid=(NBLK,),
        in_specs=[
            pl.BlockSpec((1, MLEN, SPATIAL), lambda i: (i, 0, 0)),
            pl.BlockSpec((NMEM, MLEN), lambda i: (0, 0)),
        ],
        out_specs=pl.BlockSpec((1, 1, SPATIAL), lambda i: (i, 0, 0)),
        out_shape=jax.ShapeDtypeStruct((NBLK, 1, SPATIAL), jnp.int32),
        scratch_shapes=[pltpu.VMEM((NMEM, MLEN), jnp.float32)],
    )(xr, memory)


@functools.partial(
    pl.kernel,
    mesh=plsc.VectorSubcoreMesh(core_axis_name="c", subcore_axis_name="s"),
    out_type=jax.ShapeDtypeStruct((NBLK, MLEN, SPATIAL), jnp.float32),
    scratch_types=[
        pltpu.VMEM((MLEN * NMEM,), jnp.float32),
        pltpu.VMEM((SPATIAL,), jnp.int32),
        pltpu.VMEM((MLEN, SPATIAL), jnp.float32),
    ],
    compiler_params=pltpu.CompilerParams(needs_layout_passes=False),
)
def _sc_gather(mt_hbm, idx_hbm, out_hbm, mt_v, idx_v, out_v):
    wid = lax.axis_index("s") * NCORES + lax.axis_index("c")
    pltpu.sync_copy(mt_hbm, mt_v)

    def blk_body(i, carry):
        n = wid * BLK_PER_W + i
        pltpu.sync_copy(idx_hbm.at[n], idx_v)

        def j_body(j, c2):
            civ = idx_v[pl.ds(j * LANES, LANES)]
            for k in range(MLEN):
                g = plsc.load_gather(mt_v, [civ + k * NMEM])
                out_v[k, pl.ds(j * LANES, LANES)] = g
            return c2

        lax.fori_loop(0, JSTEPS, j_body, 0)
        pltpu.sync_copy(out_v, out_hbm.at[n])
        return carry

    lax.fori_loop(0, BLK_PER_W, blk_body, 0)


@functools.partial(jax.jit, static_argnames=())
def kernel(x, memory):
    xr = x.reshape(NBLK, MLEN, SPATIAL)
    idx = _tc_indices(xr, memory)
    mt = memory.T.reshape(-1)
    out = _sc_gather(mt, idx.reshape(NBLK, SPATIAL))
    return out.reshape(B, NUM_FEAT, D1, D2)


# speedup vs baseline: 2.1558x; 1.1335x over previous
import functools

import jax
import jax.numpy as jnp
from jax import lax
from jax.experimental import pallas as pl
from jax.experimental.pallas import tpu as pltpu
from jax.experimental.pallas import tpu_sc as plsc

B = 4
NUM_FEAT = 768
D1 = 24
D2 = 24
NMEM = 1024
MLEN = 16
GROUPS = NUM_FEAT // MLEN
SPATIAL = D1 * D2
NBLK = B * GROUPS

NCORES = 2
NSUB = 16
NW = NCORES * NSUB
BLK_PER_W = NBLK // NW
LANES = 16
JSTEPS = SPATIAL // LANES


UNROLL = 2


def _mnorm_block(mem_ref, mnorm_ref):
    mem = mem_ref[...]
    nrm = jnp.sqrt(jnp.sum(mem * mem, axis=1, keepdims=True))
    mnorm_ref[...] = mem / jnp.maximum(nrm, 1e-12)


def _argmax_block(x_ref, mnorm_ref, idx_ref):
    mnorm = mnorm_ref[...]
    for u in range(UNROLL):
        X = x_ref[u]
        xnrm = jnp.sqrt(jnp.sum(X * X, axis=0, keepdims=True))
        Xn = X / jnp.maximum(xnrm, 1e-12)
        sim = lax.dot_general(
            Xn, mnorm, (((0,), (1,)), ((), ())),
            preferred_element_type=jnp.float32)
        idx_ref[u] = jnp.argmax(sim, axis=1).reshape(1, SPATIAL)


def _tc_indices(xr, memory):
    mnorm = pl.pallas_call(
        _mnorm_block,
        out_shape=jax.ShapeDtypeStruct((NMEM, MLEN), jnp.float32),
    )(memory)
    return pl.pallas_call(
        _argmax_block,
        grid=(NBLK // UNROLL,),
        in_specs=[
            pl.BlockSpec((UNROLL, MLEN, SPATIAL), lambda i: (i, 0, 0)),
            pl.BlockSpec((NMEM, MLEN), lambda i: (0, 0)),
        ],
        out_specs=pl.BlockSpec((UNROLL, 1, SPATIAL), lambda i: (i, 0, 0)),
        out_shape=jax.ShapeDtypeStruct((NBLK, 1, SPATIAL), jnp.int32),
    )(xr, mnorm)


@functools.partial(
    pl.kernel,
    mesh=plsc.VectorSubcoreMesh(core_axis_name="c", subcore_axis_name="s"),
    out_type=jax.ShapeDtypeStruct((NBLK, MLEN, SPATIAL), jnp.float32),
    scratch_types=[
        pltpu.VMEM((MLEN * NMEM,), jnp.float32),
        pltpu.VMEM((SPATIAL,), jnp.int32),
        pltpu.VMEM((MLEN, SPATIAL), jnp.float32),
    ],
    compiler_params=pltpu.CompilerParams(needs_layout_passes=False),
)
def _sc_gather(mt_hbm, idx_hbm, out_hbm, mt_v, idx_v, out_v):
    wid = lax.axis_index("s") * NCORES + lax.axis_index("c")
    pltpu.sync_copy(mt_hbm, mt_v)

    def blk_body(i, carry):
        n = wid * BLK_PER_W + i
        pltpu.sync_copy(idx_hbm.at[n], idx_v)

        def j_body(j, c2):
            civ = idx_v[pl.ds(j * LANES, LANES)]
            for k in range(MLEN):
                g = plsc.load_gather(mt_v, [civ + k * NMEM])
                out_v[k, pl.ds(j * LANES, LANES)] = g
            return c2

        lax.fori_loop(0, JSTEPS, j_body, 0)
        pltpu.sync_copy(out_v, out_hbm.at[n])
        return carry

    lax.fori_loop(0, BLK_PER_W, blk_body, 0)


@functools.partial(jax.jit, static_argnames=())
def kernel(x, memory):
    xr = x.reshape(NBLK, MLEN, SPATIAL)
    idx = _tc_indices(xr, memory)
    mt = memory.T.reshape(-1)
    out = _sc_gather(mt, idx.reshape(NBLK, SPATIAL))
    return out.reshape(B, NUM_FEAT, D1, D2)
